# Initial kernel scaffold; baseline (speedup 1.0000x reference)
import jax
import jax.numpy as jnp
from jax.experimental import pallas as pl


def kernel(vertices, features):
    raise NotImplementedError("write your pallas kernel here")


# speedup vs baseline: 1.9968x; 1.9968x over previous
import functools

import jax
import jax.numpy as jnp
from jax import lax
from jax.experimental import pallas as pl
from jax.experimental.pallas import tpu as pltpu
from jax.experimental.pallas import tpu_sc as plsc

VOL = 64
NCELLS = VOL * VOL * VOL
NPTS = 100000
NTILE = 16
NPAD = 102400
CH = 6
NGRP = 6
PC = 128
PPT = NPAD // NTILE
NCHUNK = PPT // PC
LG = PC // 16
NELEM = 8 * CH * PC
TSLICE = NCELLS // NTILE
FCH = 16
FR = TSLICE // FCH


def _sc_body(verts, feats, zeros, out, vbuf, fbuf, dbuf, ibuf, ard, col, acc):
    core = lax.axis_index("c")
    sid = lax.axis_index("s")
    iota = lax.iota(jnp.int32, 16)

    def fill_body(f, carry):
        fill = f * 2 + core
        b = fill // NGRP
        g = fill % NGRP

        pltpu.sync_copy(zeros, acc.at[pl.ds(sid * TSLICE * CH, TSLICE * CH)])
        plsc.subcore_barrier()

        def chunk_body(ci, c2):
            p0 = sid * PPT + ci * PC
            pltpu.sync_copy(verts.at[b, :, pl.ds(p0, PC)], vbuf)
            pltpu.sync_copy(feats.at[b, pl.ds(g * 8, 8), pl.ds(p0, PC)], fbuf)

            def lg_body(l, c3):
                o = l * 16
                x = vbuf[0, pl.ds(o, 16)]
                y = vbuf[1, pl.ds(o, 16)]
                z = vbuf[2, pl.ds(o, 16)]

                def prep(v):
                    s = jnp.minimum(jnp.maximum(v, -1.0), 1.0)
                    fidx = (s + 1.0) * 31.5
                    fl = jnp.minimum(fidx.astype(jnp.int32), 62)
                    w = fidx - fl.astype(jnp.float32)
                    return fl, w

                fx, wx = prep(x)
                fy, wy = prep(y)
                fz, wz = prep(z)
                cb = (fx * 4096 + fy * 64 + fz) * CH
                mwx = 1.0 - wx
                mwy = 1.0 - wy
                mwz = 1.0 - wz
                ab = (mwx * mwy, mwx * wy, wx * mwy, wx * wy)

                fv = [fbuf[c, pl.ds(o, 16)] for c in range(CH)]

                for k in range(8):
                    xi, yi, zi = (k >> 2) & 1, (k >> 1) & 1, k & 1
                    ck = cb + (xi * 4096 + yi * 64 + zi) * CH
                    wk = ab[xi * 2 + yi] * (wz if zi else mwz)
                    for c in range(CH):
                        e0 = (k * CH + c) * PC + o
                        ibuf[pl.ds(e0, 16)] = ck + c
                        dbuf[pl.ds(e0, 16)] = wk * fv[c]
                return c3

            lax.fori_loop(0, LG, lg_body, 0)
            pltpu.sync_copy(dbuf, acc.at[ibuf], add=True)
            return c2

        lax.fori_loop(0, NCHUNK, chunk_body, 0)
        plsc.subcore_barrier()

        ch0 = g * CH

        def flush_chunk(fc, c5):
            base = sid * TSLICE + fc * FR
            pltpu.sync_copy(acc.at[pl.ds(base * CH, FR * CH)], ard)
            for c in range(CH):
                cvec = jnp.full((16,), c, dtype=jnp.int32)

                def gat(i, c6):
                    v = plsc.load_gather(ard, [(i * 16 + iota) * CH + cvec])
                    col[pl.ds(i * 16, 16)] = v
                    return c6

                lax.fori_loop(0, FR // 16, gat, 0)
                ch = ch0 + c

                @pl.when(ch < 32)
                def _write():
                    off = (b * 32 + ch) * NCELLS + base
                    pltpu.sync_copy(col, out.at[pl.ds(off, FR)])
            return c5

        lax.fori_loop(0, FCH, flush_chunk, 0)
        plsc.subcore_barrier()
        return carry

    lax.fori_loop(0, NGRP, fill_body, 0)


@functools.partial(
    pl.kernel,
    out_type=jax.ShapeDtypeStruct((2 * 32 * NCELLS,), jnp.float32),
    mesh=plsc.VectorSubcoreMesh(core_axis_name="c", subcore_axis_name="s"),
    scratch_types=[
        pltpu.VMEM((3, PC), jnp.float32),
        pltpu.VMEM((8, PC), jnp.float32),
        pltpu.VMEM((NELEM,), jnp.float32),
        pltpu.VMEM((NELEM,), jnp.int32),
        pltpu.VMEM((FR * CH,), jnp.float32),
        pltpu.VMEM((FR,), jnp.float32),
        pltpu.VMEM_SHARED((NCELLS * CH,), jnp.float32),
    ],
    compiler_params=pltpu.CompilerParams(needs_layout_passes=False,
                                         use_tc_tiling_on_sc=False),
)
def _sc_project(verts, feats, zeros, out, *scratch):
    _sc_body(verts, feats, zeros, out, *scratch)


def kernel(vertices, features):
    verts_t = jnp.transpose(vertices, (0, 2, 1))
    verts_t = jnp.pad(verts_t, ((0, 0), (0, 0), (0, NPAD - NPTS)))
    feats_t = jnp.transpose(features, (0, 2, 1))
    feats_t = jnp.pad(feats_t, ((0, 0), (0, CH * NGRP - 32), (0, 0)))
    feats_t = feats_t.reshape(2, NGRP, CH, NPTS)
    feats_t = jnp.pad(feats_t,
                      ((0, 0), (0, 0), (0, 8 - CH), (0, NPAD - NPTS)))
    feats_t = feats_t.reshape(2, NGRP * 8, NPAD)
    zeros = jnp.zeros((TSLICE * CH,), jnp.float32)
    out = _sc_project(verts_t, feats_t, zeros)
    return out.reshape(2, 32, VOL, VOL, VOL)


# speedup vs baseline: 2.6733x; 1.3387x over previous
import functools

import jax
import jax.numpy as jnp
from jax import lax
from jax.experimental import pallas as pl
from jax.experimental.pallas import tpu as pltpu
from jax.experimental.pallas import tpu_sc as plsc

VOL = 64
NCELLS = VOL * VOL * VOL
NPTS = 100000
NTILE = 16
CH = 6
CHS = 2
PC = 128
PPT = NPTS // NTILE
NCHUNK = (PPT + PC - 1) // PC
LG = PC // 16
TSLICE = NCELLS // NTILE
ZSLICE = NCELLS * CH // NTILE


def _do_fill(b, ch0, chw, sid, iota, verts, feats, zeros, out,
             vbuf, fbuf, dbuf, ibuf, acc):
    nelem = 8 * chw * PC

    zw = NCELLS * chw // NTILE
    pltpu.sync_copy(zeros.at[pl.ds(0, zw)], acc.at[pl.ds(sid * zw, zw)])
    plsc.subcore_barrier()

    def chunk_body(ci, c2):
        start = ci * PC
        p0 = jnp.minimum(start, PPT - PC)
        d0 = start - p0
        pg = sid * PPT + p0
        pltpu.sync_copy(verts.at[b, pl.ds(pg, PC), :], vbuf)
        pltpu.sync_copy(feats.at[b, pl.ds(pg, PC), :], fbuf)

        def lg_body(l, c3):
            o = l * 16
            ridx = o + iota
            zero16 = jnp.zeros((16,), jnp.float32)
            one16 = jnp.full((16,), 1.0, jnp.float32)
            maskf = jnp.where(ridx >= d0, one16, zero16)
            x = plsc.load_gather(vbuf, [ridx, jnp.full((16,), 0, jnp.int32)])
            y = plsc.load_gather(vbuf, [ridx, jnp.full((16,), 1, jnp.int32)])
            z = plsc.load_gather(vbuf, [ridx, jnp.full((16,), 2, jnp.int32)])

            def prep(v):
                s = jnp.minimum(jnp.maximum(v, -1.0), 1.0)
                fidx = (s + 1.0) * 31.5
                fl = jnp.minimum(fidx.astype(jnp.int32), 62)
                w = fidx - fl.astype(jnp.float32)
                return fl, w

            fx, wx = prep(x)
            fy, wy = prep(y)
            fz, wz = prep(z)
            cb = fx * 4096 + fy * 64 + fz
            mwx = 1.0 - wx
            mwy = 1.0 - wy
            mwz = 1.0 - wz
            ab = (mwx * mwy, mwx * wy, wx * mwy, wx * wy)

            fv = [plsc.load_gather(fbuf, [ridx, jnp.full((16,), 1, jnp.int32) * (ch0 + c)])
                  for c in range(chw)]

            for k in range(8):
                xi, yi, zi = (k >> 2) & 1, (k >> 1) & 1, k & 1
                ck = cb + (xi * 4096 + yi * 64 + zi)
                wk = ab[xi * 2 + yi] * (wz if zi else mwz) * maskf
                for c in range(chw):
                    e0 = (k * chw + c) * PC + o
                    ibuf[pl.ds(e0, 16)] = ck + c * NCELLS
                    dbuf[pl.ds(e0, 16)] = wk * fv[c]
            return c3

        lax.fori_loop(0, LG, lg_body, 0)
        pltpu.sync_copy(dbuf.at[pl.ds(0, nelem)],
                        acc.at[ibuf.at[pl.ds(0, nelem)]], add=True)
        return c2

    lax.fori_loop(0, NCHUNK, chunk_body, 0)
    plsc.subcore_barrier()

    for c in range(chw):
        src0 = c * NCELLS + sid * TSLICE
        off = (b * 32 + ch0 + c) * NCELLS + sid * TSLICE
        pltpu.sync_copy(acc.at[pl.ds(src0, TSLICE)], out.at[pl.ds(off, TSLICE)])
    plsc.subcore_barrier()


def _sc_body(verts, feats, zeros, out, vbuf, fbuf, dbuf, ibuf, acc):
    core = lax.axis_index("c")
    sid = lax.axis_index("s")
    iota = lax.iota(jnp.int32, 16)

    def fill_body(f, carry):
        fill = f * 2 + core
        b = fill // 5
        g = fill % 5
        _do_fill(b, g * CH, CH, sid, iota, verts, feats, zeros, out,
                 vbuf, fbuf, dbuf, ibuf, acc)
        return carry

    lax.fori_loop(0, 5, fill_body, 0)
    _do_fill(core, 30, CHS, sid, iota, verts, feats, zeros, out,
             vbuf, fbuf, dbuf, ibuf, acc)


@functools.partial(
    pl.kernel,
    out_type=jax.ShapeDtypeStruct((2 * 32 * NCELLS,), jnp.float32),
    mesh=plsc.VectorSubcoreMesh(core_axis_name="c", subcore_axis_name="s"),
    scratch_types=[
        pltpu.VMEM((PC, 3), jnp.float32),
        pltpu.VMEM((PC, 32), jnp.float32),
        pltpu.VMEM((8 * CH * PC,), jnp.float32),
        pltpu.VMEM((8 * CH * PC,), jnp.int32),
        pltpu.VMEM_SHARED((NCELLS * CH,), jnp.float32),
    ],
    compiler_params=pltpu.CompilerParams(needs_layout_passes=False,
                                         use_tc_tiling_on_sc=False),
)
def _sc_project(verts, feats, zeros, out, *scratch):
    _sc_body(verts, feats, zeros, out, *scratch)


def kernel(vertices, features):
    zeros = jnp.zeros((ZSLICE,), jnp.float32)
    out = _sc_project(vertices, features, zeros)
    return out.reshape(2, 32, VOL, VOL, VOL)


# speedup vs baseline: 3.1987x; 1.1966x over previous
import functools

import jax
import jax.numpy as jnp
from jax import lax
from jax.experimental import pallas as pl
from jax.experimental.pallas import tpu as pltpu
from jax.experimental.pallas import tpu_sc as plsc

VOL = 64
NCELLS = VOL * VOL * VOL
NPTS = 100000
NTILE = 16
CH = 6
CHS = 2
PC = 128
PPT = NPTS // NTILE
NCHUNK = (PPT + PC - 1) // PC
NPAIR = (NCHUNK - 1) // 2
LG = PC // 16
TSLICE = NCELLS // NTILE
ZSLICE = NCELLS * CH // NTILE


def _start_in(b, ci, verts, feats, vbuf, fbuf, semv, semf):
    start = ci * PC
    p0 = jnp.minimum(start, PPT - PC)
    pg = lax.axis_index("s") * PPT + p0
    pltpu.async_copy(verts.at[b, pl.ds(pg, PC), :], vbuf, semv)
    pltpu.async_copy(feats.at[b, pl.ds(pg, PC), :], fbuf, semf)


def _wait_in(b, verts, feats, vbuf, fbuf, semv, semf):
    pltpu.make_async_copy(verts.at[b, pl.ds(0, PC), :], vbuf, semv).wait()
    pltpu.make_async_copy(feats.at[b, pl.ds(0, PC), :], fbuf, semf).wait()


def _do_chunk(b, ci, ch0, chw, iota, vbuf, fbuf, dbuf, ibuf, acc, semsa,
              semsb):
    half = 4 * chw * PC
    d0 = ci * PC - jnp.minimum(ci * PC, PPT - PC)

    def build_range(k0, k1):
        def body(l, c3):
            o = l * 16
            ridx = o + iota
            zero16 = jnp.zeros((16,), jnp.float32)
            one16 = jnp.full((16,), 1.0, jnp.float32)
            maskf = jnp.where(ridx >= d0, one16, zero16)
            x = plsc.load_gather(vbuf, [ridx, jnp.full((16,), 0, jnp.int32)])
            y = plsc.load_gather(vbuf, [ridx, jnp.full((16,), 1, jnp.int32)])
            z = plsc.load_gather(vbuf, [ridx, jnp.full((16,), 2, jnp.int32)])

            def prep(v):
                s = jnp.minimum(jnp.maximum(v, -1.0), 1.0)
                fidx = (s + 1.0) * 31.5
                fl = jnp.minimum(fidx.astype(jnp.int32), 62)
                w = fidx - fl.astype(jnp.float32)
                return fl, w

            fx, wx = prep(x)
            fy, wy = prep(y)
            fz, wz = prep(z)
            cb = fx * 4096 + fy * 64 + fz
            mwx = 1.0 - wx
            mwy = 1.0 - wy
            mwz = 1.0 - wz
            ab = (mwx * mwy, mwx * wy, wx * mwy, wx * wy)

            fv = [plsc.load_gather(
                      fbuf, [ridx, jnp.full((16,), 1, jnp.int32) * (ch0 + c)])
                  for c in range(chw)]

            for k in range(k0, k1):
                xi, yi, zi = (k >> 2) & 1, (k >> 1) & 1, k & 1
                ck = cb + (xi * 4096 + yi * 64 + zi)
                wk = ab[xi * 2 + yi] * (wz if zi else mwz) * maskf
                for c in range(chw):
                    e0 = (k * chw + c) * PC + o
                    ibuf[pl.ds(e0, 16)] = ck + c * NCELLS
                    dbuf[pl.ds(e0, 16)] = wk * fv[c]
            return c3
        lax.fori_loop(0, LG, body, 0)

    def wait_half(h0, sem):
        pltpu.make_async_copy(
            dbuf.at[pl.ds(h0, half)],
            acc.at[ibuf.at[pl.ds(h0, half)]], sem).wait()

    def start_half(h0, sem):
        pltpu.async_copy(
            dbuf.at[pl.ds(h0, half)],
            acc.at[ibuf.at[pl.ds(h0, half)]], sem, add=True)

    @pl.when(ci > 0)
    def _wa():
        wait_half(0, semsa)
    build_range(0, 4)
    start_half(0, semsa)

    @pl.when(ci > 0)
    def _wb():
        wait_half(half, semsb)
    build_range(4, 8)
    start_half(half, semsb)


def _drain_chunk(chw, dbuf, ibuf, acc, semsa, semsb):
    half = 4 * chw * PC
    for h0, sem in ((0, semsa), (half, semsb)):
        pltpu.make_async_copy(
            dbuf.at[pl.ds(h0, half)],
            acc.at[ibuf.at[pl.ds(h0, half)]], sem).wait()


def _do_fill(b, ch0, chw, sid, iota, verts, feats, zeros, out,
             vb, fb, dbuf, ibuf, acc, semv, semf, semsa, semsb):
    zw = NCELLS * chw // NTILE
    pltpu.sync_copy(zeros.at[pl.ds(0, zw)], acc.at[pl.ds(sid * zw, zw)])
    plsc.subcore_barrier()

    _start_in(b, 0, verts, feats, vb[0], fb[0], semv[0], semf[0])

    def pair_body(i, c2):
        ci0 = i * 2
        _wait_in(b, verts, feats, vb[0], fb[0], semv[0], semf[0])
        _start_in(b, ci0 + 1, verts, feats, vb[1], fb[1], semv[1], semf[1])
        _do_chunk(b, ci0, ch0, chw, iota, vb[0], fb[0], dbuf, ibuf, acc,
                  semsa, semsb)
        _wait_in(b, verts, feats, vb[1], fb[1], semv[1], semf[1])
        _start_in(b, ci0 + 2, verts, feats, vb[0], fb[0], semv[0], semf[0])
        _do_chunk(b, ci0 + 1, ch0, chw, iota, vb[1], fb[1], dbuf, ibuf, acc,
                  semsa, semsb)
        return c2

    lax.fori_loop(0, NPAIR, pair_body, 0)
    _wait_in(b, verts, feats, vb[0], fb[0], semv[0], semf[0])
    _do_chunk(b, jnp.int32(NCHUNK - 1), ch0, chw, iota, vb[0], fb[0],
              dbuf, ibuf, acc, semsa, semsb)
    _drain_chunk(chw, dbuf, ibuf, acc, semsa, semsb)
    plsc.subcore_barrier()

    for c in range(chw):
        src0 = c * NCELLS + sid * TSLICE
        off = (b * 32 + ch0 + c) * NCELLS + sid * TSLICE
        pltpu.sync_copy(acc.at[pl.ds(src0, TSLICE)], out.at[pl.ds(off, TSLICE)])
    plsc.subcore_barrier()


def _sc_body(verts, feats, zeros, out, vb0, fb0, vb1, fb1, dbuf, ibuf, acc,
             semv0, semf0, semv1, semf1, semsa, semsb):
    core = lax.axis_index("c")
    sid = lax.axis_index("s")
    iota = lax.iota(jnp.int32, 16)
    vb = (vb0, vb1)
    fb = (fb0, fb1)
    semv = (semv0, semv1)
    semf = (semf0, semf1)

    def fill_body(f, carry):
        fill = f * 2 + core
        b = fill // 5
        g = fill % 5
        _do_fill(b, g * CH, CH, sid, iota, verts, feats, zeros, out,
                 vb, fb, dbuf, ibuf, acc, semv, semf, semsa, semsb)
        return carry

    lax.fori_loop(0, 5, fill_body, 0)
    _do_fill(core, 30, CHS, sid, iota, verts, feats, zeros, out,
             vb, fb, dbuf, ibuf, acc, semv, semf, semsa, semsb)


@functools.partial(
    pl.kernel,
    out_type=jax.ShapeDtypeStruct((2 * 32 * NCELLS,), jnp.float32),
    mesh=plsc.VectorSubcoreMesh(core_axis_name="c", subcore_axis_name="s"),
    scratch_types=[
        pltpu.VMEM((PC, 3), jnp.float32),
        pltpu.VMEM((PC, 32), jnp.float32),
        pltpu.VMEM((PC, 3), jnp.float32),
        pltpu.VMEM((PC, 32), jnp.float32),
        pltpu.VMEM((8 * CH * PC,), jnp.float32),
        pltpu.VMEM((8 * CH * PC,), jnp.int32),
        pltpu.VMEM_SHARED((NCELLS * CH,), jnp.float32),
        pltpu.SemaphoreType.DMA,
        pltpu.SemaphoreType.DMA,
        pltpu.SemaphoreType.DMA,
        pltpu.SemaphoreType.DMA,
        pltpu.SemaphoreType.DMA,
        pltpu.SemaphoreType.DMA,
    ],
    compiler_params=pltpu.CompilerParams(needs_layout_passes=False,
                                         use_tc_tiling_on_sc=False),
)
def _sc_project(verts, feats, zeros, out, *scratch):
    _sc_body(verts, feats, zeros, out, *scratch)


def kernel(vertices, features):
    zeros = jnp.zeros((ZSLICE,), jnp.float32)
    out = _sc_project(vertices, features, zeros)
    return out.reshape(2, 32, VOL, VOL, VOL)


# speedup vs baseline: 3.2694x; 1.0221x over previous
import functools

import jax
import jax.numpy as jnp
from jax import lax
from jax.experimental import pallas as pl
from jax.experimental.pallas import tpu as pltpu
from jax.experimental.pallas import tpu_sc as plsc

VOL = 64
NCELLS = VOL * VOL * VOL
NPTS = 100000
NTILE = 16
CH = 6
CHS = 2
PC = 176
PPT = NPTS // NTILE
NCHUNK = (PPT + PC - 1) // PC
NPAIR = (NCHUNK - 1) // 2
NLEFT = (NCHUNK - 1) - 2 * NPAIR
LG = PC // 16
TSLICE = NCELLS // NTILE
ZSLICE = NCELLS * CH // NTILE


def _start_in(b, ci, ch0, chw, verts, feats, vbuf, fbuf, semv, semf):
    start = ci * PC
    p0 = jnp.minimum(start, PPT - PC)
    pg = lax.axis_index("s") * PPT + p0
    pltpu.async_copy(verts.at[b, pl.ds(pg, PC), :], vbuf, semv)
    pltpu.async_copy(feats.at[b, pl.ds(pg, PC), :], fbuf, semf)


def _wait_in(b, chw, verts, feats, vbuf, fbuf, semv, semf):
    pltpu.make_async_copy(verts.at[b, pl.ds(0, PC), :], vbuf, semv).wait()
    pltpu.make_async_copy(feats.at[b, pl.ds(0, PC), :], fbuf, semf).wait()


def _do_chunk(b, ci, ch0, chw, iota, vbuf, fbuf, dbuf, ibuf, acc, semsa,
              semsb):
    half = 4 * chw * PC
    d0 = ci * PC - jnp.minimum(ci * PC, PPT - PC)

    def build_range(k0, k1):
        def body(l, c3):
            o = l * 16
            ridx = o + iota
            zero16 = jnp.zeros((16,), jnp.float32)
            one16 = jnp.full((16,), 1.0, jnp.float32)
            maskf = jnp.where(ridx >= d0, one16, zero16)
            x = plsc.load_gather(vbuf, [ridx, jnp.full((16,), 0, jnp.int32)])
            y = plsc.load_gather(vbuf, [ridx, jnp.full((16,), 1, jnp.int32)])
            z = plsc.load_gather(vbuf, [ridx, jnp.full((16,), 2, jnp.int32)])

            def prep(v):
                s = jnp.minimum(jnp.maximum(v, -1.0), 1.0)
                fidx = (s + 1.0) * 31.5
                fl = jnp.minimum(fidx.astype(jnp.int32), 62)
                w = fidx - fl.astype(jnp.float32)
                return fl, w

            fx, wx = prep(x)
            fy, wy = prep(y)
            fz, wz = prep(z)
            cb = fx * 4096 + fy * 64 + fz
            mwx = 1.0 - wx
            mwy = 1.0 - wy
            mwz = 1.0 - wz
            ab = (mwx * mwy, mwx * wy, wx * mwy, wx * wy)

            fv = [plsc.load_gather(
                      fbuf, [ridx, jnp.full((16,), 1, jnp.int32) * (ch0 + c)])
                  for c in range(chw)]

            for k in range(k0, k1):
                xi, yi, zi = (k >> 2) & 1, (k >> 1) & 1, k & 1
                ck = cb + (xi * 4096 + yi * 64 + zi)
                wk = ab[xi * 2 + yi] * (wz if zi else mwz) * maskf
                for c in range(chw):
                    e0 = (k * chw + c) * PC + o
                    ibuf[pl.ds(e0, 16)] = ck + c * NCELLS
                    dbuf[pl.ds(e0, 16)] = wk * fv[c]
            return c3
        lax.fori_loop(0, LG, body, 0)

    def wait_half(h0, sem):
        pltpu.make_async_copy(
            dbuf.at[pl.ds(h0, half)],
            acc.at[ibuf.at[pl.ds(h0, half)]], sem).wait()

    def start_half(h0, sem):
        pltpu.async_copy(
            dbuf.at[pl.ds(h0, half)],
            acc.at[ibuf.at[pl.ds(h0, half)]], sem, add=True)

    @pl.when(ci > 0)
    def _wa():
        wait_half(0, semsa)
    build_range(0, 4)
    start_half(0, semsa)

    @pl.when(ci > 0)
    def _wb():
        wait_half(half, semsb)
    build_range(4, 8)
    start_half(half, semsb)


def _drain_chunk(chw, dbuf, ibuf, acc, semsa, semsb):
    half = 4 * chw * PC
    for h0, sem in ((0, semsa), (half, semsb)):
        pltpu.make_async_copy(
            dbuf.at[pl.ds(h0, half)],
            acc.at[ibuf.at[pl.ds(h0, half)]], sem).wait()


def _do_fill(b, ch0, chw, sid, iota, verts, feats, zeros, out,
             vb, fb, dbuf, ibuf, acc, semv, semf, semsa, semsb):
    zw = NCELLS * chw // NTILE
    pltpu.sync_copy(zeros.at[pl.ds(0, zw)], acc.at[pl.ds(sid * zw, zw)])
    plsc.subcore_barrier()

    _start_in(b, 0, ch0, chw, verts, feats, vb[0], fb[0], semv[0], semf[0])

    def pair_body(i, c2):
        ci0 = i * 2
        _wait_in(b, chw, verts, feats, vb[0], fb[0], semv[0], semf[0])
        _start_in(b, ci0 + 1, ch0, chw, verts, feats, vb[1], fb[1],
                  semv[1], semf[1])
        _do_chunk(b, ci0, ch0, chw, iota, vb[0], fb[0], dbuf, ibuf, acc,
                  semsa, semsb)
        _wait_in(b, chw, verts, feats, vb[1], fb[1], semv[1], semf[1])
        _start_in(b, ci0 + 2, ch0, chw, verts, feats, vb[0], fb[0],
                  semv[0], semf[0])
        _do_chunk(b, ci0 + 1, ch0, chw, iota, vb[1], fb[1], dbuf, ibuf, acc,
                  semsa, semsb)
        return c2

    lax.fori_loop(0, NPAIR, pair_body, 0)
    if NLEFT:
        _wait_in(b, chw, verts, feats, vb[0], fb[0], semv[0], semf[0])
        _start_in(b, NCHUNK - 1, ch0, chw, verts, feats, vb[1], fb[1],
                  semv[1], semf[1])
        _do_chunk(b, jnp.int32(NCHUNK - 2), ch0, chw, iota, vb[0], fb[0],
                  dbuf, ibuf, acc, semsa, semsb)
        _wait_in(b, chw, verts, feats, vb[1], fb[1], semv[1], semf[1])
        _do_chunk(b, jnp.int32(NCHUNK - 1), ch0, chw, iota, vb[1], fb[1],
                  dbuf, ibuf, acc, semsa, semsb)
    else:
        _wait_in(b, chw, verts, feats, vb[0], fb[0], semv[0], semf[0])
        _do_chunk(b, jnp.int32(NCHUNK - 1), ch0, chw, iota, vb[0], fb[0],
                  dbuf, ibuf, acc, semsa, semsb)
    _drain_chunk(chw, dbuf, ibuf, acc, semsa, semsb)
    plsc.subcore_barrier()

    for c in range(chw):
        src0 = c * NCELLS + sid * TSLICE
        off = (b * 32 + ch0 + c) * NCELLS + sid * TSLICE
        pltpu.sync_copy(acc.at[pl.ds(src0, TSLICE)], out.at[pl.ds(off, TSLICE)])
    plsc.subcore_barrier()


def _sc_body(verts, feats, zeros, out, vb0, fb0, vb1, fb1, dbuf, ibuf, acc,
             semv0, semf0, semv1, semf1, semsa, semsb):
    core = lax.axis_index("c")
    sid = lax.axis_index("s")
    iota = lax.iota(jnp.int32, 16)
    vb = (vb0, vb1)
    fb = (fb0, fb1)
    semv = (semv0, semv1)
    semf = (semf0, semf1)

    def fill_body(f, carry):
        fill = f * 2 + core
        b = fill // 5
        g = fill % 5
        _do_fill(b, g * CH, CH, sid, iota, verts, feats, zeros, out,
                 vb, fb, dbuf, ibuf, acc, semv, semf, semsa, semsb)
        return carry

    lax.fori_loop(0, 5, fill_body, 0)
    _do_fill(core, 30, CHS, sid, iota, verts, feats, zeros, out,
             vb, fb, dbuf, ibuf, acc, semv, semf, semsa, semsb)


@functools.partial(
    pl.kernel,
    out_type=jax.ShapeDtypeStruct((2 * 32 * NCELLS,), jnp.float32),
    mesh=plsc.VectorSubcoreMesh(core_axis_name="c", subcore_axis_name="s"),
    scratch_types=[
        pltpu.VMEM((PC, 3), jnp.float32),
        pltpu.VMEM((PC, 32), jnp.float32),
        pltpu.VMEM((PC, 3), jnp.float32),
        pltpu.VMEM((PC, 32), jnp.float32),
        pltpu.VMEM((8 * CH * PC,), jnp.float32),
        pltpu.VMEM((8 * CH * PC,), jnp.int32),
        pltpu.VMEM_SHARED((NCELLS * CH,), jnp.float32),
        pltpu.SemaphoreType.DMA,
        pltpu.SemaphoreType.DMA,
        pltpu.SemaphoreType.DMA,
        pltpu.SemaphoreType.DMA,
        pltpu.SemaphoreType.DMA,
        pltpu.SemaphoreType.DMA,
    ],
    compiler_params=pltpu.CompilerParams(needs_layout_passes=False,
                                         use_tc_tiling_on_sc=False),
)
def _sc_project(verts, feats, zeros, out, *scratch):
    _sc_body(verts, feats, zeros, out, *scratch)


def kernel(vertices, features):
    zeros = jnp.zeros((ZSLICE,), jnp.float32)
    out = _sc_project(vertices, features, zeros)
    return out.reshape(2, 32, VOL, VOL, VOL)
